# baseline (speedup 1.0000x reference)
import functools

import jax
import jax.numpy as jnp
from jax import lax
from jax.experimental import pallas as pl
from jax.experimental.pallas import tpu as pltpu
from jax.experimental.pallas import tpu_sc as plsc

C = 128
K = 16
KK = K + 1
N = 10000
NW = 32
NS = 16
N_PAD = 10240
NBLK = 2048
NB = N_PAD // NBLK
SLOW_C = 0
PF = 640
PS = 0
FTOT = NS * PF
OSUB = 80


def _tc_tables_body(y_ref, w_ref, b_ref, z_ref):
    z = jax.lax.dot_general(
        y_ref[...], w_ref[0],
        (((0,), (0,)), ((), ())),
        preferred_element_type=jnp.float32,
    )
    z_ref[0] = z + b_ref[0]


def _build_tables(y2, wall, ball):
    return pl.pallas_call(
        _tc_tables_body,
        grid=(NB, KK),
        in_specs=[
            pl.BlockSpec((C, NBLK), lambda nb, g: (0, nb)),
            pl.BlockSpec((1, C, C), lambda nb, g: (g, 0, 0)),
            pl.BlockSpec((1, 1, C), lambda nb, g: (g, 0, 0)),
        ],
        out_specs=pl.BlockSpec((1, NBLK, C), lambda nb, g: (g, nb, 0)),
        out_shape=jax.ShapeDtypeStruct((KK, N_PAD, C), jnp.float32),
    )(y2, wall, ball)


def _segs(total):
    out, o = [], 0
    while o < total:
        s = min(128, total - o)
        out.append((o, s))
        o += s
    return tuple(out)


@functools.partial(
    pl.kernel,
    out_type=jax.ShapeDtypeStruct((N, C), jnp.float32),
    mesh=plsc.VectorSubcoreMesh(core_axis_name="c", subcore_axis_name="s"),
    scratch_types=[
        pltpu.VMEM((K, PF), jnp.int32),
        pltpu.VMEM((K, 128), jnp.int32),
        pltpu.VMEM((PF, C), jnp.float32),
        pltpu.SemaphoreType.DMA,
        pltpu.SemaphoreType.DMA,
    ],
)
def _sc_gather_sum(ztab, idxw_f, out, idxt_v, idxs_v, acc_v, gsem, bsem):
    cc = lax.axis_index("c")
    sid = lax.axis_index("s")

    def run(start, count, guarded):
        iref = idxt_v
        pltpu.async_copy(idxw_f.at[sid], idxt_v, bsem)
        idx_cp = pltpu.make_async_copy(idxw_f.at[sid], idxt_v, bsem)
        pltpu.async_copy(ztab.at[pl.ds(K * N_PAD + start, count)],
                         acc_v.at[pl.ds(0, count)], gsem)
        idx_cp.wait()
        pltpu.make_async_copy(ztab.at[pl.ds(K * N_PAD + start, count)],
                              acc_v.at[pl.ds(0, count)], gsem).wait()
        for g in range(K):
            for o, s in _segs(count):
                pltpu.async_copy(ztab.at[iref.at[g, pl.ds(o, s)]],
                                 acc_v.at[pl.ds(o, s)], gsem, add=True)
        for g in range(K):
            for o, s in _segs(count):
                pltpu.make_async_copy(ztab.at[iref.at[g, pl.ds(o, s)]],
                                      acc_v.at[pl.ds(o, s)], gsem).wait()
        if not guarded:
            pltpu.sync_copy(acc_v.at[pl.ds(0, count)],
                            out.at[pl.ds(start, count)])
        else:
            for j in range(count // OSUB):
                @pl.when(start + (j + 1) * OSUB <= N)
                def _(j=j):
                    pltpu.sync_copy(acc_v.at[pl.ds(j * OSUB, OSUB)],
                                    out.at[pl.ds(start + j * OSUB, OSUB)])

    @pl.when(cc != SLOW_C)
    def _():
        run(sid * PF, PF, guarded=True)


def kernel(x, y, y_xyz, params, idx):
    p = params
    y2 = y[0]
    wv3 = p['Wv'].reshape(C, C, K)
    a = jnp.transpose(wv3, (2, 1, 0))
    wall = jnp.concatenate([a, -a.sum(axis=0, keepdims=True)], axis=0)
    ball = jnp.zeros((KK, 1, C), jnp.float32).at[K, 0].set(p['bv'])

    idx2 = idx[0].astype(jnp.int32)
    idxp = jnp.zeros((N_PAD, K), jnp.int32).at[:N].set(idx2)
    offs = (jnp.arange(K, dtype=jnp.int32) * N_PAD)[None, :, None]
    idxw_f = jnp.transpose(idxp[:FTOT].reshape(NS, PF, K), (0, 2, 1)) + offs

    zall = _build_tables(y2, wall, ball)
    ztab = zall.reshape(KK * N_PAD, C)

    out_rows = _sc_gather_sum(ztab, idxw_f)
    return out_rows.T[None]
